# baseline (speedup 1.0000x reference)
import dataclasses

import jax
import jax.numpy as jnp
import numpy as np
from jax import lax
from jax.experimental import pallas as pl
from jax.experimental.pallas import tpu as pltpu
from jax.experimental.pallas import tpu_sc as plsc

TEMP = np.float32(0.7)
TOPP = np.float32(0.9)
K = 50
ROWS = 128
VOCAB = 100000
CHUNK = 400
CHUNK_V = CHUNK // 16
NCHUNK = VOCAB // CHUNK
NCH_PAD = 256
CAND_CAP = 2048
NTILES = 32
ROWS_PER = ROWS // NTILES
HOT = 64

TC_BLK = 3200
TC_GRID = 32

NEG = np.float32(-np.inf)


def _neg_vec():
    return jnp.full((16,), NEG, jnp.float32)


def _scalar_store(ref, idx, val, lanes):
    g = (idx // 16) * 16
    v = ref[pl.ds(g, 16)]
    ref[pl.ds(g, 16)] = jnp.where(lanes == idx - g, val, v)


def _splat(x):
    return x if getattr(x, "ndim", 0) else jnp.full((16,), x)


def _tc_chunkmax_body(x_ref, o_ref):
    i = pl.program_id(0)
    x = x_ref[...]
    lcol = lax.broadcasted_iota(jnp.int32, (ROWS, TC_BLK), 1)
    gcol = lcol + i * TC_BLK
    xm = jnp.where(gcol < VOCAB, x, NEG)
    outs = []
    for c in range(8):
        sel = jnp.logical_and(lcol >= c * CHUNK, lcol < (c + 1) * CHUNK)
        outs.append(jnp.max(jnp.where(sel, xm, NEG), axis=1, keepdims=True))
    o_ref[...] = jnp.concatenate(outs, axis=1)[None]


def _tc_chunkmax(logits):
    out = pl.pallas_call(
        _tc_chunkmax_body,
        grid=(TC_GRID,),
        in_specs=[pl.BlockSpec((ROWS, TC_BLK), lambda i: (0, i))],
        out_specs=pl.BlockSpec((1, ROWS, 8), lambda i: (i, 0, 0)),
        out_shape=jax.ShapeDtypeStruct((TC_GRID, ROWS, 8), jnp.float32),
    )(logits)
    return out.transpose(1, 0, 2).reshape(ROWS, NCH_PAD)


def _sc_body(chunks_hbm, chmax_hbm, noise_hbm, out_hbm,
             hotbuf, cand, chmax, hotlist, topbuf, noisebuf, cdfbuf, outbuf,
             sem):
    wid = lax.axis_index("s") * 2 + lax.axis_index("c")
    lanes = lax.iota(jnp.int32, 16)
    falsev = lanes < 0

    outbuf[pl.ds(0, 16)] = jnp.zeros((16,), jnp.float32)
    hotlist[pl.ds(48, 16)] = jnp.zeros((16,), jnp.int32)

    @pl.loop(0, ROWS_PER)
    def _row(i):
        r = wid * ROWS_PER + i
        pltpu.sync_copy(noise_hbm.at[r], noisebuf)
        pltpu.sync_copy(chmax_hbm.at[r], chmax)

        cvecs = tuple(chmax[pl.ds(j * 16, 16)] for j in range(16))
        rbase = r * NCHUNK

        def ext_thresh(t, carry):
            vs = list(carry[:16])
            ws = list(vs)
            while len(ws) > 1:
                ws = [jnp.maximum(ws[k], ws[k + 1]) for k in range(0, len(ws) - 1, 2)] \
                     + ([ws[-1]] if len(ws) % 2 else [])
            s = jnp.max(ws[0])

            done = falsev
            idacc = jnp.zeros((16,), jnp.int32)
            out = []
            for j in range(16):
                f = _splat(plsc.all_reduce_ffs(vs[j] == s))
                hit = f < 16
                sel = jnp.logical_and(jnp.logical_not(done), hit)
                out.append(jnp.where(jnp.logical_and(sel, lanes == f), NEG, vs[j]))
                idacc = jnp.where(sel, rbase + jnp.int32(16 * j) + f, idacc)
                done = jnp.logical_or(done, hit)

            g = (t // 16) * 16
            hv = hotlist[pl.ds(g, 16)]
            hotlist[pl.ds(g, 16)] = jnp.where(lanes == t - g, idacc, hv)
            return tuple(out) + (s,)

        res = lax.fori_loop(0, K, ext_thresh, cvecs + (NEG,))
        t0 = res[16]

        pltpu.async_copy(chunks_hbm.at[hotlist], hotbuf, sem).wait()

        def chunk_body(t, cnt):
            for j in range(CHUNK_V):
                v = hotbuf[t, pl.ds(j * 16, 16)]
                msk = v >= t0
                mi = jnp.where(msk, jnp.int32(1), jnp.int32(0))
                pos = plsc.cumsum(mi) - mi
                idx = jnp.minimum(cnt + pos, CAND_CAP - 1)
                plsc.store_scatter(cand, [idx], v, mask=msk)
                cnt = cnt + _splat(plsc.all_reduce_population_count(msk))
            return cnt

        cnt_v = lax.fori_loop(0, K, chunk_body, jnp.zeros((16,), jnp.int32))
        cnt = jnp.minimum(jnp.max(cnt_v), CAND_CAP - 16)
        cand[pl.ds(cnt, 16)] = _neg_vec()
        nv = (cnt + 15) // 16

        topbuf[pl.ds(48, 16)] = _neg_vec()

        def ext_top(t, _):
            def mx(j, m):
                return jnp.maximum(m, cand[pl.ds(j * 16, 16)])
            m = lax.fori_loop(0, nv, mx, _neg_vec())
            s = jnp.max(m)
            _scalar_store(topbuf, t, s, lanes)

            def clr(j, done):
                v = cand[pl.ds(j * 16, 16)]
                f = _splat(plsc.all_reduce_ffs(v == s))
                hit = f < 16
                sel = jnp.logical_and(jnp.logical_not(done), hit)
                cand[pl.ds(j * 16, 16)] = jnp.where(
                    jnp.logical_and(sel, lanes == f), NEG, v)
                return jnp.logical_or(done, hit)

            lax.fori_loop(0, nv, clr, falsev)
            return 0

        lax.fori_loop(0, K, ext_top, 0)

        vvecs = []
        pvecs = []
        for j in range(4):
            v = topbuf[pl.ds(j * 16, 16)] / TEMP
            vvecs.append(v)
            if j == 0:
                vmax = v[0]
            pvecs.append(jnp.exp(v - vmax))
        z = jnp.sum(pvecs[0] + pvecs[1] + pvecs[2] + pvecs[3])

        carry = jnp.float32(0)
        for j in range(4):
            c = plsc.cumsum(pvecs[j] / z) + carry
            cdfbuf[pl.ds(1 + j * 16, 16)] = c
            carry = jnp.max(c)

        smax = NEG
        svecs = []
        for j in range(4):
            sh = cdfbuf[pl.ds(j * 16, 16)]
            g = noisebuf[pl.ds(j * 16, 16)]
            mk = sh < TOPP
            if j == 0:
                mk = jnp.logical_or(mk, lanes == 0)
            sc = jnp.where(mk, vvecs[j] + g, NEG)
            svecs.append(sc)
            smax = jnp.maximum(smax, jnp.max(sc))

        w = jnp.int32(9999)
        for j in range(4):
            fs = jnp.min(_splat(plsc.all_reduce_ffs(svecs[j] == smax)))
            idx = jnp.where(fs < 16, jnp.int32(j * 16) + fs, jnp.int32(9999))
            w = jnp.minimum(w, idx)

        g0 = (w // 16) * 16
        vw = topbuf[pl.ds(g0, 16)] / TEMP
        _scalar_store(outbuf, i, jnp.max(jnp.where(lanes == w - g0, vw, NEG)), lanes)

    pltpu.sync_copy(outbuf, out_hbm.at[wid])


@jax.jit
def kernel(logits):
    noise = jax.random.gumbel(jax.random.key(42), (ROWS, K), jnp.float32)
    noise = jnp.concatenate([noise, jnp.zeros((ROWS, 14), jnp.float32)], axis=-1)
    chmax = _tc_chunkmax(logits)
    chunks = logits.reshape(ROWS * NCHUNK, CHUNK)
    mesh = plsc.VectorSubcoreMesh(core_axis_name="c", subcore_axis_name="s")
    cp = pltpu.CompilerParams()
    if "needs_layout_passes" in pltpu.CompilerParams.__dataclass_fields__:
        cp = dataclasses.replace(cp, needs_layout_passes=False)
    if "use_tc_tiling_on_sc" in pltpu.CompilerParams.__dataclass_fields__:
        cp = dataclasses.replace(cp, use_tc_tiling_on_sc=False)
    fn = pl.kernel(
        _sc_body,
        out_type=jax.ShapeDtypeStruct((NTILES, 16), jnp.float32),
        mesh=mesh,
        compiler_params=cp,
        scratch_types=[
            pltpu.VMEM((HOT, CHUNK), jnp.float32),
            pltpu.VMEM((CAND_CAP,), jnp.float32),
            pltpu.VMEM((NCH_PAD,), jnp.float32),
            pltpu.VMEM((HOT,), jnp.int32),
            pltpu.VMEM((64,), jnp.float32),
            pltpu.VMEM((64,), jnp.float32),
            pltpu.VMEM((80,), jnp.float32),
            pltpu.VMEM((16,), jnp.float32),
            pltpu.SemaphoreType.DMA,
        ],
    )
    res = fn(chunks, chmax, noise)
    return res[:, :ROWS_PER].reshape(ROWS, 1)


# speedup vs baseline: 2.0703x; 2.0703x over previous
import dataclasses

import jax
import jax.numpy as jnp
import numpy as np
from jax import lax
from jax.experimental import pallas as pl
from jax.experimental.pallas import tpu as pltpu
from jax.experimental.pallas import tpu_sc as plsc

TEMP = np.float32(0.7)
TOPP = np.float32(0.9)
K = 50
ROWS = 128
VOCAB = 100000
CHUNK = 400
CHUNK_V = CHUNK // 16
NCHUNK = VOCAB // CHUNK
CAND_CAP = 2048
NTILES = 32
ROWS_PER = ROWS // NTILES

NEG = np.float32(-np.inf)


def _neg_vec():
    return jnp.full((16,), NEG, jnp.float32)


def _scalar_store(ref, idx, val, lanes):
    g = (idx // 16) * 16
    v = ref[pl.ds(g, 16)]
    ref[pl.ds(g, 16)] = jnp.where(lanes == idx - g, val, v)


def _splat(x):
    return x if getattr(x, "ndim", 0) else jnp.full((16,), x)


def _sc_body(logits_hbm, noise_hbm, out_hbm,
             rowbuf, cand, chmax, hotlist, topbuf, noisebuf, cdfbuf, outbuf,
             sem):
    wid = lax.axis_index("s") * 2 + lax.axis_index("c")
    lanes = lax.iota(jnp.int32, 16)
    falsev = lanes < 0

    outbuf[pl.ds(0, 16)] = jnp.zeros((16,), jnp.float32)

    @pl.loop(0, ROWS_PER)
    def _row(i):
        r = wid * ROWS_PER + i
        pltpu.sync_copy(logits_hbm.at[r], rowbuf)
        pltpu.sync_copy(noise_hbm.at[r], noisebuf)

        chmax[pl.ds(240, 16)] = _neg_vec()

        @pl.loop(0, NCHUNK)
        def _ch(c):
            vs = [rowbuf[pl.ds(c * CHUNK + j * 16, 16)] for j in range(CHUNK_V)]
            while len(vs) > 1:
                vs = [jnp.maximum(vs[k], vs[k + 1]) for k in range(0, len(vs) - 1, 2)] \
                     + ([vs[-1]] if len(vs) % 2 else [])
            _scalar_store(chmax, c, jnp.max(vs[0]), lanes)

        cvecs = tuple(chmax[pl.ds(j * 16, 16)] for j in range(16))

        def ext_thresh(t, carry):
            vs = list(carry[:16])
            ws = list(vs)
            while len(ws) > 1:
                ws = [jnp.maximum(ws[k], ws[k + 1]) for k in range(0, len(ws) - 1, 2)] \
                     + ([ws[-1]] if len(ws) % 2 else [])
            s = jnp.max(ws[0])

            done = falsev
            idacc = jnp.zeros((16,), jnp.int32)
            out = []
            for j in range(16):
                f = _splat(plsc.all_reduce_ffs(vs[j] == s))
                hit = f < 16
                sel = jnp.logical_and(jnp.logical_not(done), hit)
                out.append(jnp.where(jnp.logical_and(sel, lanes == f), NEG, vs[j]))
                idacc = jnp.where(sel, jnp.int32(16 * j) + f, idacc)
                done = jnp.logical_or(done, hit)

            g = (t // 16) * 16
            hv = hotlist[pl.ds(g, 16)]
            hotlist[pl.ds(g, 16)] = jnp.where(lanes == t - g, idacc, hv)
            return tuple(out) + (s,)

        res = lax.fori_loop(0, K, ext_thresh, cvecs + (NEG,))
        t0 = res[16]

        def chunk_body(t, cnt):
            g = (t // 16) * 16
            hv = hotlist[pl.ds(g, 16)]
            cid = jnp.max(jnp.where(lanes == t - g, hv, 0))
            base = cid * CHUNK
            for lo in (0, 8, 16):
                hi = min(lo + 8, CHUNK_V)
                vs, ms, ps, ns = [], [], [], []
                for j in range(lo, hi):
                    v = rowbuf[pl.ds(base + j * 16, 16)]
                    msk = v >= t0
                    mi = jnp.where(msk, jnp.int32(1), jnp.int32(0))
                    vs.append(v)
                    ms.append(msk)
                    ps.append(plsc.cumsum(mi) - mi)
                    ns.append(_splat(plsc.all_reduce_population_count(msk)))
                for k in range(hi - lo):
                    idx = jnp.minimum(cnt + ps[k], CAND_CAP - 1)
                    plsc.store_scatter(cand, [idx], vs[k], mask=ms[k])
                    cnt = cnt + ns[k]
            return cnt

        cnt_v = lax.fori_loop(0, K, chunk_body, jnp.zeros((16,), jnp.int32))
        cnt = jnp.minimum(jnp.max(cnt_v), CAND_CAP - 16)
        cand[pl.ds(cnt, 16)] = _neg_vec()
        nv = (cnt + 15) // 16

        topbuf[pl.ds(48, 16)] = _neg_vec()

        def ext_top(t, _):
            def mx(j, m):
                return jnp.maximum(m, cand[pl.ds(j * 16, 16)])
            m = lax.fori_loop(0, nv, mx, _neg_vec())
            s = jnp.max(m)
            _scalar_store(topbuf, t, s, lanes)

            def clr(j, done):
                v = cand[pl.ds(j * 16, 16)]
                f = _splat(plsc.all_reduce_ffs(v == s))
                hit = f < 16
                sel = jnp.logical_and(jnp.logical_not(done), hit)
                cand[pl.ds(j * 16, 16)] = jnp.where(
                    jnp.logical_and(sel, lanes == f), NEG, v)
                return jnp.logical_or(done, hit)

            lax.fori_loop(0, nv, clr, falsev)
            return 0

        lax.fori_loop(0, K, ext_top, 0)

        vvecs = []
        pvecs = []
        for j in range(4):
            v = topbuf[pl.ds(j * 16, 16)] / TEMP
            vvecs.append(v)
            if j == 0:
                vmax = v[0]
            pvecs.append(jnp.exp(v - vmax))
        z = jnp.sum(pvecs[0] + pvecs[1] + pvecs[2] + pvecs[3])

        carry = jnp.float32(0)
        for j in range(4):
            c = plsc.cumsum(pvecs[j] / z) + carry
            cdfbuf[pl.ds(1 + j * 16, 16)] = c
            carry = jnp.max(c)

        smax = NEG
        svecs = []
        for j in range(4):
            sh = cdfbuf[pl.ds(j * 16, 16)]
            g = noisebuf[pl.ds(j * 16, 16)]
            mk = sh < TOPP
            if j == 0:
                mk = jnp.logical_or(mk, lanes == 0)
            sc = jnp.where(mk, vvecs[j] + g, NEG)
            svecs.append(sc)
            smax = jnp.maximum(smax, jnp.max(sc))

        w = jnp.int32(9999)
        for j in range(4):
            fs = jnp.min(_splat(plsc.all_reduce_ffs(svecs[j] == smax)))
            idx = jnp.where(fs < 16, jnp.int32(j * 16) + fs, jnp.int32(9999))
            w = jnp.minimum(w, idx)

        g0 = (w // 16) * 16
        vw = topbuf[pl.ds(g0, 16)] / TEMP
        _scalar_store(outbuf, i, jnp.max(jnp.where(lanes == w - g0, vw, NEG)), lanes)

    pltpu.sync_copy(outbuf, out_hbm.at[wid])


@jax.jit
def kernel(logits):
    noise = jax.random.gumbel(jax.random.key(42), (ROWS, K), jnp.float32)
    noise = jnp.concatenate([noise, jnp.zeros((ROWS, 14), jnp.float32)], axis=-1)
    mesh = plsc.VectorSubcoreMesh(core_axis_name="c", subcore_axis_name="s")
    cp = pltpu.CompilerParams()
    if "needs_layout_passes" in pltpu.CompilerParams.__dataclass_fields__:
        cp = dataclasses.replace(cp, needs_layout_passes=False)
    fn = pl.kernel(
        _sc_body,
        out_type=jax.ShapeDtypeStruct((NTILES, 16), jnp.float32),
        mesh=mesh,
        compiler_params=cp,
        scratch_types=[
            pltpu.VMEM((VOCAB,), jnp.float32),
            pltpu.VMEM((CAND_CAP,), jnp.float32),
            pltpu.VMEM((256,), jnp.float32),
            pltpu.VMEM((64,), jnp.int32),
            pltpu.VMEM((64,), jnp.float32),
            pltpu.VMEM((64,), jnp.float32),
            pltpu.VMEM((80,), jnp.float32),
            pltpu.VMEM((16,), jnp.float32),
            pltpu.SemaphoreType.DMA,
        ],
    )
    res = fn(logits, noise)
    return res[:, :ROWS_PER].reshape(ROWS, 1)
